# Initial kernel scaffold; baseline (speedup 1.0000x reference)
import jax
import jax.numpy as jnp
from jax.experimental import pallas as pl


def kernel(semantic_emb, collab_emb, params):
    raise NotImplementedError("write your pallas kernel here")


# speedup vs baseline: 1.8393x; 1.8393x over previous
import functools

import jax
import jax.numpy as jnp
from jax.experimental import pallas as pl

_HI = jax.lax.Precision.HIGHEST


def _mm(a, b):
    return jnp.dot(a, b, preferred_element_type=jnp.float32)


def _mm_exact(a, b):
    return jnp.dot(a, b, preferred_element_type=jnp.float32, precision=_HI)


def _ln(x, g, b, eps=1e-5):
    m = jnp.mean(x, axis=-1, keepdims=True)
    v = jnp.mean((x - m) ** 2, axis=-1, keepdims=True)
    return (x - m) / jnp.sqrt(v + eps) * g + b


def _fwd_kernel(sem_ref, col_ref,
                gw1_ref, gb1_ref, glng_ref, glnb_ref, gw2_ref, gb2_ref,
                flng_ref, flnb_ref,
                enc0_ref, enc1_ref, enc2_ref, enc3_ref,
                dec0_ref, dec1_ref, dec2_ref,
                shw1_ref, shb1_ref, shlng_ref, shlnb_ref, shw2_ref, shb2_ref,
                chw1_ref, chb1_ref, chlng_ref, chlnb_ref, chw2_ref, chb2_ref,
                cb_ref,
                sem_out_ref, col_out_ref,
                *, sd, cd, nl, ne, ld):
    sem = sem_ref[...]
    col = col_ref[...]
    tb = sem.shape[0]
    d_in = sd + cd

    h = _mm(col, gw1_ref[...]) + gb1_ref[...]
    h = jax.nn.relu(_ln(h, glng_ref[...], glnb_ref[...]))
    gate = jax.nn.sigmoid(_mm(h, gw2_ref[...]) + gb2_ref[...])
    den = gate * col

    m = (jnp.sum(sem, axis=-1, keepdims=True)
         + jnp.sum(den, axis=-1, keepdims=True)) * (1.0 / d_in)
    v = (jnp.sum((sem - m) ** 2, axis=-1, keepdims=True)
         + jnp.sum((den - m) ** 2, axis=-1, keepdims=True)) * (1.0 / d_in)
    rstd = 1.0 / jnp.sqrt(v + 1e-5)
    flng = flng_ref[...]
    flnb = flnb_ref[...]
    sem_n = (sem - m) * rstd * flng[:, :sd] + flnb[:, :sd]
    den_n = (den - m) * rstd * flng[:, sd:] + flnb[:, sd:]

    enc0 = enc0_ref[...]
    z = jax.nn.silu(_mm(sem_n, enc0[:sd, :]) + _mm(den_n, enc0[sd:, :]))
    z = jax.nn.silu(_mm(z, enc1_ref[...]))
    z = jax.nn.silu(_mm(z, enc2_ref[...]))
    z = _mm(z, enc3_ref[...])

    iota = jax.lax.broadcasted_iota(jnp.int32, (tb, ne), 1)
    residual = z
    zq = jnp.zeros_like(z)
    for l in range(nl):
        cb = cb_ref[l]
        csum = jnp.sum(cb * cb, axis=-1)[None, :]
        rsum = jnp.sum(residual * residual, axis=-1, keepdims=True)
        d = rsum - 2.0 * _mm(residual, cb.T) + csum
        dmin = jnp.min(d, axis=-1, keepdims=True)
        idx = jnp.min(jnp.where(d == dmin, iota, ne), axis=-1, keepdims=True)
        onehot = (iota == idx).astype(jnp.float32)
        q = _mm_exact(onehot, cb)
        zq = zq + q
        residual = residual - q

    h = jax.nn.silu(_mm(zq, dec0_ref[...]))
    h = jax.nn.silu(_mm(h, dec1_ref[...]))
    h = jax.nn.silu(_mm(h, dec2_ref[...]))

    s = _mm(h, shw1_ref[...]) + shb1_ref[...]
    s = jax.nn.relu(_ln(s, shlng_ref[...], shlnb_ref[...]))
    sem_out_ref[...] = _mm(s, shw2_ref[...]) + shb2_ref[...]

    c = _mm(h, chw1_ref[...]) + chb1_ref[...]
    c = jax.nn.relu(_ln(c, chlng_ref[...], chlnb_ref[...]))
    col_out_ref[...] = _mm(c, chw2_ref[...]) + chb2_ref[...]


def kernel(semantic_emb, collab_emb, params):
    p = params
    b, sd = semantic_emb.shape
    cd = collab_emb.shape[1]
    nl, ne, ld = p['codebook'].shape
    tb = 1024
    grid = (b // tb,)

    def row(name):
        return p[name].reshape(1, -1)

    weights = [
        p['gw1'], row('gb1'), row('gln_g'), row('gln_b'), p['gw2'], row('gb2'),
        row('fln_g'), row('fln_b'),
        p['enc0'], p['enc1'], p['enc2'], p['enc3'],
        p['dec0'], p['dec1'], p['dec2'],
        p['sh_w1'], row('sh_b1'), row('sh_ln_g'), row('sh_ln_b'),
        p['sh_w2'], row('sh_b2'),
        p['ch_w1'], row('ch_b1'), row('ch_ln_g'), row('ch_ln_b'),
        p['ch_w2'], row('ch_b2'),
        p['codebook'],
    ]

    def wspec(a):
        shape = a.shape
        return pl.BlockSpec(shape, lambda i, _n=len(shape): (0,) * _n)

    in_specs = [
        pl.BlockSpec((tb, sd), lambda i: (i, 0)),
        pl.BlockSpec((tb, cd), lambda i: (i, 0)),
    ] + [wspec(a) for a in weights]

    out_specs = [
        pl.BlockSpec((tb, sd), lambda i: (i, 0)),
        pl.BlockSpec((tb, cd), lambda i: (i, 0)),
    ]

    out_shape = [
        jax.ShapeDtypeStruct((b, sd), jnp.float32),
        jax.ShapeDtypeStruct((b, cd), jnp.float32),
    ]

    fn = functools.partial(_fwd_kernel, sd=sd, cd=cd, nl=nl, ne=ne, ld=ld)
    sem_out, col_out = pl.pallas_call(
        fn,
        grid=grid,
        in_specs=in_specs,
        out_specs=out_specs,
        out_shape=out_shape,
    )(semantic_emb, collab_emb, *weights)
    return sem_out, col_out
